# Initial kernel scaffold; baseline (speedup 1.0000x reference)
import jax
import jax.numpy as jnp
from jax.experimental import pallas as pl


def kernel(X_in, ones):
    raise NotImplementedError("write your pallas kernel here")


# speedup vs baseline: 1.0975x; 1.0975x over previous
import functools

import jax
import jax.numpy as jnp
from jax import lax
from jax.experimental import pallas as pl
from jax.experimental.pallas import tpu as pltpu
from jax.experimental.pallas import tpu_sc as plsc

DEPTH = 1000
BATCH = 16384

_NC = 2
_NS = 16
_L = 16
_NW = _NC * _NS
_BPW = BATCH // _NW
_C = 64
_NCHUNK = _BPW // _C
_CW = _C * DEPTH


def _onehot_body(idx_hbm, out_hbm, idx_v, buf_v, sem):
    wid = lax.axis_index("s") * _NC + lax.axis_index("c")
    base = wid * _BPW

    pltpu.sync_copy(idx_hbm.at[pl.ds(base * 1, _BPW)], idx_v)

    zeros16 = jnp.zeros((_L,), jnp.float32)

    def _zero(i, carry):
        for u in range(8):
            buf_v[pl.ds(i * 128 + u * 16, 16)] = zeros16
        return carry

    lax.fori_loop(0, _CW // 128, _zero, 0)

    lanes = lax.broadcasted_iota(jnp.int32, (_L,), 0)
    ones16 = jnp.ones((_L,), jnp.float32)

    for c in range(_NCHUNK):
        for v in range(_C // _L):
            iv = idx_v[pl.ds(c * _C + v * _L, _L)]
            flat = (lanes + (v * _L)) * DEPTH + iv
            plsc.store_scatter(buf_v, [flat], ones16)
        pltpu.sync_copy(buf_v, out_hbm.at[pl.ds((base + c * _C) * DEPTH, _CW)])
        for v in range(_C // _L):
            iv = idx_v[pl.ds(c * _C + v * _L, _L)]
            flat = (lanes + (v * _L)) * DEPTH + iv
            plsc.store_scatter(buf_v, [flat], zeros16)


@jax.jit
def _onehot_sc(idx):
    mesh = plsc.VectorSubcoreMesh(core_axis_name="c", subcore_axis_name="s")
    fn = functools.partial(
        pl.kernel,
        mesh=mesh,
        out_type=jax.ShapeDtypeStruct((BATCH * DEPTH,), jnp.float32),
        scratch_types=[
            pltpu.VMEM((_BPW,), jnp.int32),
            pltpu.VMEM((_CW,), jnp.float32),
            pltpu.SemaphoreType.DMA,
        ],
        compiler_params=pltpu.CompilerParams(needs_layout_passes=False),
    )(_onehot_body)
    return fn(idx)


def kernel(X_in, ones):
    del ones
    idx = X_in.astype(jnp.int32)
    flat = _onehot_sc(idx)
    return flat.reshape(BATCH, DEPTH)


# speedup vs baseline: 1.7610x; 1.6045x over previous
import functools

import jax
import jax.numpy as jnp
from jax import lax
from jax.experimental import pallas as pl
from jax.experimental.pallas import tpu as pltpu
from jax.experimental.pallas import tpu_sc as plsc

DEPTH = 1000
BATCH = 16384

_NC = 2
_NS = 16
_L = 16
_NW = _NC * _NS
_BPW = BATCH // _NW
_C = 64
_NCHUNK = _BPW // _C


def _onehot_body(idx_hbm, out_hbm, idx_v, buf_v):
    wid = lax.axis_index("s") * _NC + lax.axis_index("c")
    base = wid * _BPW

    pltpu.sync_copy(idx_hbm.at[pl.ds(base, _BPW)], idx_v)

    zeros16 = jnp.zeros((_L,), jnp.float32)

    def _zero_row(r, carry):
        for k in range(DEPTH // _L):
            buf_v[r, pl.ds(k * _L, _L)] = zeros16
        buf_v[r, pl.ds(DEPTH - _L, _L)] = zeros16
        return carry

    lax.fori_loop(0, _C, _zero_row, 0)

    lanes = lax.broadcasted_iota(jnp.int32, (_L,), 0)
    ones16 = jnp.ones((_L,), jnp.float32)

    for c in range(_NCHUNK):
        for v in range(_C // _L):
            iv = idx_v[pl.ds(c * _C + v * _L, _L)]
            rows = lanes + (v * _L)
            plsc.store_scatter(buf_v, [rows, iv], ones16)
        pltpu.sync_copy(buf_v, out_hbm.at[pl.ds(base + c * _C, _C)])
        for v in range(_C // _L):
            iv = idx_v[pl.ds(c * _C + v * _L, _L)]
            rows = lanes + (v * _L)
            plsc.store_scatter(buf_v, [rows, iv], zeros16)


@jax.jit
def _onehot_sc(idx):
    mesh = plsc.VectorSubcoreMesh(core_axis_name="c", subcore_axis_name="s")
    fn = functools.partial(
        pl.kernel,
        mesh=mesh,
        out_type=jax.ShapeDtypeStruct((BATCH, DEPTH), jnp.float32),
        scratch_types=[
            pltpu.VMEM((_BPW,), jnp.int32),
            pltpu.VMEM((_C, DEPTH), jnp.float32),
        ],
        compiler_params=pltpu.CompilerParams(needs_layout_passes=False),
    )(_onehot_body)
    return fn(idx)


def kernel(X_in, ones):
    del ones
    idx = X_in.astype(jnp.int32)
    return _onehot_sc(idx)
